# baseline (device time: 8116 ns/iter reference)
import jax
import jax.numpy as jnp
from jax import lax
from jax.experimental import pallas as pl
from jax.experimental.pallas import tpu as pltpu

N_CHUNK = 4


def kernel(x):
    m, n = x.shape
    rows = m // 128
    m_per = m // N_CHUNK
    rows_per = rows // N_CHUNK

    def body(x_hbm, out_ref, xbuf, send_buf, recv_buf, in_sems, send_sem, recv_sem):
        my_x = lax.axis_index("x")
        my_y = lax.axis_index("y")
        nbr = (my_x, 1 - my_y)

        barrier_sem = pltpu.get_barrier_semaphore()
        pl.semaphore_signal(
            barrier_sem, inc=1, device_id=nbr,
            device_id_type=pl.DeviceIdType.MESH,
        )

        def chunk_copy(c, slot):
            return pltpu.make_async_copy(
                x_hbm.at[pl.ds(c * m_per, m_per), :],
                xbuf.at[slot],
                in_sems.at[slot],
            )

        chunk_copy(0, 0).start()
        for c in range(N_CHUNK):
            slot = c % 2
            if c + 1 < N_CHUNK:
                chunk_copy(c + 1, (c + 1) % 2).start()
            chunk_copy(c, slot).wait()
            send_buf[c * rows_per : (c + 1) * rows_per, :] = jnp.sum(
                xbuf[slot].reshape(rows_per, 128, n), axis=2
            )

        pl.semaphore_wait(barrier_sem, 1)
        rdma = pltpu.make_async_remote_copy(
            src_ref=send_buf,
            dst_ref=recv_buf,
            send_sem=send_sem,
            recv_sem=recv_sem,
            device_id=nbr,
            device_id_type=pl.DeviceIdType.MESH,
        )
        rdma.start()
        rdma.wait()

        out_ref[:, :] = send_buf[:, :] + recv_buf[:, :]

    out = pl.pallas_call(
        body,
        out_shape=jax.ShapeDtypeStruct((rows, 128), jnp.float32),
        in_specs=[pl.BlockSpec(memory_space=pl.ANY)],
        out_specs=pl.BlockSpec(memory_space=pltpu.VMEM),
        scratch_shapes=[
            pltpu.VMEM((2, m_per, n), jnp.float32),
            pltpu.VMEM((rows, 128), jnp.float32),
            pltpu.VMEM((rows, 128), jnp.float32),
            pltpu.SemaphoreType.DMA((2,)),
            pltpu.SemaphoreType.DMA,
            pltpu.SemaphoreType.DMA,
        ],
        compiler_params=pltpu.CompilerParams(collective_id=0),
    )(x)
    return out.reshape(m, 1)


# device time: 7187 ns/iter; 1.1293x vs baseline; 1.1293x over previous
import jax
import jax.numpy as jnp
from jax import lax
from jax.experimental import pallas as pl
from jax.experimental.pallas import tpu as pltpu

N_CHUNK = 4


def kernel(x):
    m, n = x.shape
    rows = m // 128
    m_per = m // N_CHUNK
    rows_per = rows // N_CHUNK

    def body(x_ref, out_ref, send_buf, recv_buf, send_sem, recv_sem):
        c = pl.program_id(0)
        my_x = lax.axis_index("x")
        my_y = lax.axis_index("y")
        nbr = (my_x, 1 - my_y)
        barrier_sem = pltpu.get_barrier_semaphore()

        @pl.when(c == 0)
        def _():
            pl.semaphore_signal(
                barrier_sem, inc=1, device_id=nbr,
                device_id_type=pl.DeviceIdType.MESH,
            )

        send_buf[pl.ds(c * rows_per, rows_per), :] = jnp.sum(
            x_ref[:, :].reshape(rows_per, 128, n), axis=2
        )

        @pl.when(c == N_CHUNK - 1)
        def _():
            pl.semaphore_wait(barrier_sem, 1)
            rdma = pltpu.make_async_remote_copy(
                src_ref=send_buf,
                dst_ref=recv_buf,
                send_sem=send_sem,
                recv_sem=recv_sem,
                device_id=nbr,
                device_id_type=pl.DeviceIdType.MESH,
            )
            rdma.start()
            rdma.wait()
            out_ref[:, :] = send_buf[:, :] + recv_buf[:, :]

    out = pl.pallas_call(
        body,
        grid=(N_CHUNK,),
        out_shape=jax.ShapeDtypeStruct((rows, 128), jnp.float32),
        in_specs=[
            pl.BlockSpec((m_per, n), lambda c: (c, 0)),
        ],
        out_specs=pl.BlockSpec((rows, 128), lambda c: (0, 0)),
        scratch_shapes=[
            pltpu.VMEM((rows, 128), jnp.float32),
            pltpu.VMEM((rows, 128), jnp.float32),
            pltpu.SemaphoreType.DMA,
            pltpu.SemaphoreType.DMA,
        ],
        compiler_params=pltpu.CompilerParams(collective_id=0),
    )(x)
    return out.reshape(m, 1)


# device time: 6889 ns/iter; 1.1781x vs baseline; 1.0433x over previous
import jax
import jax.numpy as jnp
from jax import lax
from jax.experimental import pallas as pl
from jax.experimental.pallas import tpu as pltpu


def kernel(x):
    m, n = x.shape
    rows = m // 128

    def body(x_ref, out_ref, send_buf, recv_buf, send_sem, recv_sem):
        my_x = lax.axis_index("x")
        my_y = lax.axis_index("y")
        nbr = (my_x, 1 - my_y)

        barrier_sem = pltpu.get_barrier_semaphore()
        pl.semaphore_signal(
            barrier_sem, inc=1, device_id=nbr,
            device_id_type=pl.DeviceIdType.MESH,
        )

        send_buf[:, :] = jnp.sum(x_ref[:, :].reshape(rows, 128, n), axis=2)

        pl.semaphore_wait(barrier_sem, 1)

        rdma = pltpu.make_async_remote_copy(
            src_ref=send_buf,
            dst_ref=recv_buf,
            send_sem=send_sem,
            recv_sem=recv_sem,
            device_id=nbr,
            device_id_type=pl.DeviceIdType.MESH,
        )
        rdma.start()
        rdma.wait()

        out_ref[:, :] = send_buf[:, :] + recv_buf[:, :]

    out = pl.pallas_call(
        body,
        out_shape=jax.ShapeDtypeStruct((rows, 128), jnp.float32),
        in_specs=[pl.BlockSpec(memory_space=pltpu.VMEM)],
        out_specs=pl.BlockSpec(memory_space=pltpu.VMEM),
        scratch_shapes=[
            pltpu.VMEM((rows, 128), jnp.float32),
            pltpu.VMEM((rows, 128), jnp.float32),
            pltpu.SemaphoreType.DMA,
            pltpu.SemaphoreType.DMA,
        ],
        compiler_params=pltpu.CompilerParams(collective_id=0),
    )(x)
    return out.reshape(m, 1)
